# baseline (speedup 1.0000x reference)
import jax
import jax.numpy as jnp
from jax import lax
from jax.experimental import pallas as pl
from jax.experimental.pallas import tpu as pltpu
from jax.experimental.pallas import tpu_sc as plsc

B, N, D = 4, 8192, 2048
H = N // 2
NC, NS = 2, 16
NW = NC * NS
TOT = B * N
RW = TOT // NW
CH = 16


NBUF = 3
NSTEPS = RW // CH


def _body(in_hbm, out_hbm, bufs, idxs, rsems, wsems):
    wid = lax.axis_index("s") * NC + lax.axis_index("c")
    b = wid // 8
    j = (wid // 4) % 2
    q = wid % 4
    in_base = b * N + j * H + q * RW
    o_pat = b * N + j + 2 * (q * RW) + 2 * lax.iota(jnp.int32, 16)

    def rd(it, p):
        pltpu.async_copy(in_hbm.at[pl.ds(in_base + it * CH, CH), :],
                         bufs.at[p], rsems.at[p])

    def wr(it, p):
        idxs[p] = o_pat + 2 * (it * CH)
        pltpu.async_copy(bufs.at[p], out_hbm.at[idxs.at[p]], wsems.at[p])

    def wait_rd(p):
        pltpu.make_async_copy(in_hbm.at[pl.ds(in_base, CH), :], bufs.at[p],
                              rsems.at[p]).wait()

    def wait_wr(p):
        pltpu.make_async_copy(bufs.at[p], out_hbm.at[idxs.at[p]],
                              wsems.at[p]).wait()

    rd(0, 0)
    rd(1, 1)
    for it in range(NSTEPS):
        p = it % NBUF
        wait_rd(p)
        wr(it, p)
        nxt = it + NBUF - 1
        if nxt < NSTEPS:
            if nxt >= NBUF:
                wait_wr(nxt % NBUF)
            rd(nxt, nxt % NBUF)
    for p in range(NBUF):
        wait_wr(p)


@jax.jit
def kernel(inputs):
    mesh = plsc.VectorSubcoreMesh(
        core_axis_name="c", subcore_axis_name="s", num_cores=NC,
        num_subcores=NS)
    out = pl.kernel(
        _body,
        out_type=jax.ShapeDtypeStruct((TOT, D), jnp.float32),
        mesh=mesh,
        scratch_types=[
            pltpu.VMEM((NBUF, CH, D), jnp.float32),
            pltpu.VMEM((NBUF, 16), jnp.int32),
            pltpu.SemaphoreType.DMA((NBUF,)),
            pltpu.SemaphoreType.DMA((NBUF,)),
        ],
    )(inputs.reshape(TOT, D))
    return out.reshape(B, N, D)


# speedup vs baseline: 1.0069x; 1.0069x over previous
import jax
import jax.numpy as jnp
from jax import lax
from jax.experimental import pallas as pl
from jax.experimental.pallas import tpu as pltpu
from jax.experimental.pallas import tpu_sc as plsc

B, N, D = 4, 8192, 2048
H = N // 2
NC, NS = 2, 16
NW = NC * NS
TOT = B * N
RW = TOT // NW
CH = 16


NBUF = 3
NSTEPS = RW // CH


def _body(in_hbm, out_hbm, bufs, idxs, rsems, wsems):
    wid = lax.axis_index("s") * NC + lax.axis_index("c")
    o_base = wid * RW
    b = o_base // N
    i0_base = (o_base % N) // 2
    k = lax.iota(jnp.int32, 16)
    pattern = (k >> 1) + (k & 1) * H + b * N

    def rd(it, p):
        idxs[p] = pattern + (i0_base + it * (CH // 2))
        pltpu.async_copy(in_hbm.at[idxs.at[p]], bufs.at[p], rsems.at[p])

    def wr(it, p):
        pltpu.async_copy(bufs.at[p],
                         out_hbm.at[pl.ds(o_base + it * CH, CH), :],
                         wsems.at[p])

    def wait_rd(p):
        pltpu.make_async_copy(in_hbm.at[idxs.at[p]], bufs.at[p],
                              rsems.at[p]).wait()

    def wait_wr(p):
        pltpu.make_async_copy(bufs.at[p],
                              out_hbm.at[pl.ds(o_base, CH), :],
                              wsems.at[p]).wait()

    rd(0, 0)
    rd(1, 1)
    for it in range(NSTEPS):
        p = it % NBUF
        wait_rd(p)
        wr(it, p)
        nxt = it + NBUF - 1
        if nxt < NSTEPS:
            if nxt >= NBUF:
                wait_wr(nxt % NBUF)
            rd(nxt, nxt % NBUF)
    for p in range(NBUF):
        wait_wr(p)


@jax.jit
def kernel(inputs):
    mesh = plsc.VectorSubcoreMesh(
        core_axis_name="c", subcore_axis_name="s", num_cores=NC,
        num_subcores=NS)
    out = pl.kernel(
        _body,
        out_type=jax.ShapeDtypeStruct((TOT, D), jnp.float32),
        mesh=mesh,
        scratch_types=[
            pltpu.VMEM((NBUF, CH, D), jnp.float32),
            pltpu.VMEM((NBUF, 16), jnp.int32),
            pltpu.SemaphoreType.DMA((NBUF,)),
            pltpu.SemaphoreType.DMA((NBUF,)),
        ],
    )(inputs.reshape(TOT, D))
    return out.reshape(B, N, D)
